# baseline (speedup 1.0000x reference)
import jax
import jax.numpy as jnp
from jax.experimental import pallas as pl
from jax.experimental.pallas import tpu as pltpu

GRID = (48, 48)
NCELL = GRID[0] * GRID[1]
NA = 6
NB = 4096
BLK = 512
FMIN = jnp.finfo(jnp.float32).min


def _body(mask_ref, logits_ref, conv_ref, out_ref):
    logits = jnp.broadcast_to(logits_ref[...], (3 * NA, NCELL))
    conv = jnp.broadcast_to(conv_ref[...], (3 * NA, NCELL))
    cls = jax.lax.broadcasted_iota(jnp.int32, (3 * NA, NCELL), 0)
    onehot = conv == jnp.where(cls >= NA, jnp.where(cls >= 2 * NA, cls - 2 * NA,
                                                    cls - NA), cls)
    hi = logits.astype(jnp.bfloat16).astype(jnp.float32)
    val = jnp.where(cls < NA, hi, jnp.where(cls < 2 * NA, logits - hi, 1.0))
    wt = jnp.where(onehot, val, 0.0).astype(jnp.bfloat16)

    maskb = mask_ref[...].reshape(NCELL, BLK).astype(jnp.bfloat16)
    acc = jax.lax.dot_general(
        wt, maskb, (((1,), (0,)), ((), ())),
        preferred_element_type=jnp.float32)

    sums = acc[:NA, :] + acc[NA:2 * NA, :]
    counts = acc[2 * NA:, :]
    total = jnp.sum(counts, axis=0, keepdims=True)
    row = jax.lax.broadcasted_iota(jnp.int32, sums.shape, 0)
    scaled = jnp.where(row == NA - 1, sums * (1.0 / 225.0), sums)
    out = jnp.where(counts > 0.5, scaled, FMIN)
    out_ref[...] = jnp.where((total < 0.5) & (row == 0), 1.0, out)


def kernel(logits, monoaction_mask, monofield_base_converter):
    mask_t = monoaction_mask.transpose(1, 2, 0).view(jnp.int8)
    out_t = pl.pallas_call(
        _body,
        grid=(NB // BLK,),
        in_specs=[
            pl.BlockSpec((GRID[0], GRID[1], BLK), lambda i: (0, 0, i)),
            pl.BlockSpec((1, NCELL), lambda i: (0, 0)),
            pl.BlockSpec((1, NCELL), lambda i: (0, 0)),
        ],
        out_specs=pl.BlockSpec((NA, BLK), lambda i: (0, i)),
        out_shape=jax.ShapeDtypeStruct((NA, NB), jnp.float32),
    )(mask_t, logits.reshape(1, NCELL),
      monofield_base_converter.reshape(1, NCELL))
    return out_t.T


# speedup vs baseline: 1.1058x; 1.1058x over previous
import jax
import jax.numpy as jnp
from jax.experimental import pallas as pl
from jax.experimental.pallas import tpu as pltpu

GRID = (48, 48)
NCELL = GRID[0] * GRID[1]
NA = 6
NB = 4096
BLK = 2048
FMIN = jnp.finfo(jnp.float32).min


def _body(mask_ref, logits_ref, conv_ref, out_ref):
    logits = jnp.broadcast_to(logits_ref[...], (3 * NA, NCELL))
    conv = jnp.broadcast_to(conv_ref[...], (3 * NA, NCELL))
    cls = jax.lax.broadcasted_iota(jnp.int32, (3 * NA, NCELL), 0)
    onehot = conv == jnp.where(cls >= NA, jnp.where(cls >= 2 * NA, cls - 2 * NA,
                                                    cls - NA), cls)
    hi = logits.astype(jnp.bfloat16).astype(jnp.float32)
    val = jnp.where(cls < NA, hi, jnp.where(cls < 2 * NA, logits - hi, 1.0))
    wt = jnp.where(onehot, val, 0.0).astype(jnp.bfloat16)

    maskb = mask_ref[...].reshape(NCELL, BLK).astype(jnp.bfloat16)
    acc = jax.lax.dot_general(
        wt, maskb, (((1,), (0,)), ((), ())),
        preferred_element_type=jnp.float32)

    sums = acc[:NA, :] + acc[NA:2 * NA, :]
    counts = acc[2 * NA:, :]
    total = jnp.sum(counts, axis=0, keepdims=True)
    row = jax.lax.broadcasted_iota(jnp.int32, sums.shape, 0)
    scaled = jnp.where(row == NA - 1, sums * (1.0 / 225.0), sums)
    out = jnp.where(counts > 0.5, scaled, FMIN)
    out_ref[...] = jnp.where((total < 0.5) & (row == 0), 1.0, out)


def kernel(logits, monoaction_mask, monofield_base_converter):
    mask_t = monoaction_mask.transpose(1, 2, 0).view(jnp.int8)
    out_t = pl.pallas_call(
        _body,
        grid=(NB // BLK,),
        in_specs=[
            pl.BlockSpec((GRID[0], GRID[1], BLK), lambda i: (0, 0, i)),
            pl.BlockSpec((1, NCELL), lambda i: (0, 0)),
            pl.BlockSpec((1, NCELL), lambda i: (0, 0)),
        ],
        out_specs=pl.BlockSpec((NA, BLK), lambda i: (0, i)),
        out_shape=jax.ShapeDtypeStruct((NA, NB), jnp.float32),
    )(mask_t, logits.reshape(1, NCELL),
      monofield_base_converter.reshape(1, NCELL))
    return out_t.T
